# Initial kernel scaffold; baseline (speedup 1.0000x reference)
import jax
import jax.numpy as jnp
from jax.experimental import pallas as pl


def kernel(x, pos, edge_index, W1a, b1a, W1b, b1b, W2a, b2a, W2b, b2b, W3a, b3a, W3b, b3b):
    raise NotImplementedError("write your pallas kernel here")


# speedup vs baseline: 1.1946x; 1.1946x over previous
import functools

import jax
import jax.numpy as jnp
from jax.experimental import pallas as pl

N = 10000
E = 160000


def _dense_node_kernel(f_ref, pos_ref, waxT_ref, wapT_ref, ba_ref, wbT_ref,
                       bb_ref, u_ref, v_ref, s_ref):
    a = jnp.dot(f_ref[...], waxT_ref[...],
                preferred_element_type=jnp.float32) + ba_ref[...]
    p = jnp.dot(pos_ref[...], wapT_ref[...],
                preferred_element_type=jnp.float32)
    u_ref[...] = a + p
    v_ref[...] = p
    s_ref[...] = jnp.dot(jax.nn.relu(a), wbT_ref[...],
                         preferred_element_type=jnp.float32) + bb_ref[...]


def _edge_mlp_kernel(us_ref, vd_ref, wbT_ref, bb_ref, m_ref):
    t = jax.nn.relu(us_ref[...] - vd_ref[...])
    m_ref[...] = jnp.dot(t, wbT_ref[...],
                         preferred_element_type=jnp.float32) + bb_ref[...]


def _dense_node(f, pos, waxT, wapT, ba, wbT, bb):
    n, din = f.shape
    h = waxT.shape[1]
    bn = 1000
    grid = (n // bn,)
    out_sds = jax.ShapeDtypeStruct((n, h), jnp.float32)
    u, v, s = pl.pallas_call(
        _dense_node_kernel,
        grid=grid,
        in_specs=[
            pl.BlockSpec((bn, din), lambda i: (i, 0)),
            pl.BlockSpec((bn, 3), lambda i: (i, 0)),
            pl.BlockSpec((din, h), lambda i: (0, 0)),
            pl.BlockSpec((3, h), lambda i: (0, 0)),
            pl.BlockSpec((1, h), lambda i: (0, 0)),
            pl.BlockSpec((h, h), lambda i: (0, 0)),
            pl.BlockSpec((1, h), lambda i: (0, 0)),
        ],
        out_specs=[
            pl.BlockSpec((bn, h), lambda i: (i, 0)),
            pl.BlockSpec((bn, h), lambda i: (i, 0)),
            pl.BlockSpec((bn, h), lambda i: (i, 0)),
        ],
        out_shape=[out_sds, out_sds, out_sds],
    )(f, pos, waxT, wapT, ba, wbT, bb)
    return u, v, s


def _edge_mlp(us, vd, wbT, bb):
    e, h = us.shape
    be = 2000
    grid = (e // be,)
    m = pl.pallas_call(
        _edge_mlp_kernel,
        grid=grid,
        in_specs=[
            pl.BlockSpec((be, h), lambda i: (i, 0)),
            pl.BlockSpec((be, h), lambda i: (i, 0)),
            pl.BlockSpec((h, h), lambda i: (0, 0)),
            pl.BlockSpec((1, h), lambda i: (0, 0)),
        ],
        out_specs=pl.BlockSpec((be, h), lambda i: (i, 0)),
        out_shape=jax.ShapeDtypeStruct((e, h), jnp.float32),
    )(us, vd, wbT, bb)
    return m


def _layer(f, pos, src, dst, Wa, ba, Wb, bb):
    din = f.shape[1]
    waxT = Wa[:, :din].T
    wapT = Wa[:, din:].T
    wbT = Wb.T
    u, v, s = _dense_node(f, pos, waxT, wapT, ba.reshape(1, -1), wbT,
                          bb.reshape(1, -1))
    us = jnp.take(u, src, axis=0)
    vd = jnp.take(v, dst, axis=0)
    m = _edge_mlp(us, vd, wbT, bb.reshape(1, -1))
    agg = jax.ops.segment_max(m, dst, num_segments=f.shape[0])
    return jnp.maximum(agg, s)


def kernel(x, pos, edge_index, W1a, b1a, W1b, b1b, W2a, b2a, W2b, b2b,
           W3a, b3a, W3b, b3b):
    src = edge_index[0]
    dst = edge_index[1]
    h = _layer(x, pos, src, dst, W1a, b1a, W1b, b1b)
    h = _layer(h, pos, src, dst, W2a, b2a, W2b, b2b)
    h = _layer(h, pos, src, dst, W3a, b3a, W3b, b3b)
    return h


# speedup vs baseline: 2.2248x; 1.8623x over previous
import functools

import jax
import jax.numpy as jnp
from jax import lax
from jax.experimental import pallas as pl
from jax.experimental.pallas import tpu as pltpu
from jax.experimental.pallas import tpu_sc as plsc

N = 10000
E = 160000
NC = 2
NS = 16
NW = NC * NS


def _dense_node_kernel(f_ref, pos_ref, waxT_ref, wapT_ref, ba_ref, wbT_ref,
                       bb_ref, u_ref, v_ref, s_ref):
    a = jnp.dot(f_ref[...], waxT_ref[...],
                preferred_element_type=jnp.float32) + ba_ref[...]
    p = jnp.dot(pos_ref[...], wapT_ref[...],
                preferred_element_type=jnp.float32)
    u_ref[...] = a + p
    v_ref[...] = p
    s_ref[...] = jnp.dot(jax.nn.relu(a), wbT_ref[...],
                         preferred_element_type=jnp.float32) + bb_ref[...]


def _edge_mlp_kernel(us_ref, vd_ref, wbT_ref, bb_ref, m_ref):
    t = jax.nn.relu(us_ref[...] - vd_ref[...])
    m_ref[...] = jnp.dot(t, wbT_ref[...],
                         preferred_element_type=jnp.float32) + bb_ref[...]


def _dense_node(f, pos, waxT, wapT, ba, wbT, bb):
    n, din = f.shape
    hu = waxT.shape[1]
    hs = wbT.shape[1]
    bn = 1000
    grid = (n // bn,)
    uv_sds = jax.ShapeDtypeStruct((n, hu), jnp.float32)
    s_sds = jax.ShapeDtypeStruct((n, hs), jnp.float32)
    u, v, s = pl.pallas_call(
        _dense_node_kernel,
        grid=grid,
        in_specs=[
            pl.BlockSpec((bn, din), lambda i: (i, 0)),
            pl.BlockSpec((bn, 3), lambda i: (i, 0)),
            pl.BlockSpec((din, hu), lambda i: (0, 0)),
            pl.BlockSpec((3, hu), lambda i: (0, 0)),
            pl.BlockSpec((1, hu), lambda i: (0, 0)),
            pl.BlockSpec((hu, hs), lambda i: (0, 0)),
            pl.BlockSpec((1, hs), lambda i: (0, 0)),
        ],
        out_specs=[
            pl.BlockSpec((bn, hu), lambda i: (i, 0)),
            pl.BlockSpec((bn, hu), lambda i: (i, 0)),
            pl.BlockSpec((bn, hs), lambda i: (i, 0)),
        ],
        out_shape=[uv_sds, uv_sds, s_sds],
    )(f, pos, waxT, wapT, ba, wbT, bb)
    return u, v, s


def _edge_mlp(us, vd, wbT, bb):
    e, hu = us.shape
    hs = wbT.shape[1]
    be = 2000
    grid = (e // be,)
    m = pl.pallas_call(
        _edge_mlp_kernel,
        grid=grid,
        in_specs=[
            pl.BlockSpec((be, hu), lambda i: (i, 0)),
            pl.BlockSpec((be, hu), lambda i: (i, 0)),
            pl.BlockSpec((hu, hs), lambda i: (0, 0)),
            pl.BlockSpec((1, hs), lambda i: (0, 0)),
        ],
        out_specs=pl.BlockSpec((be, hs), lambda i: (i, 0)),
        out_shape=jax.ShapeDtypeStruct((e, hs), jnp.float32),
    )(us, vd, wbT, bb)
    return m


def _edge_gather(u, v, src, dst):
    h = u.shape[1]
    per_w = E // NW
    cg = 200
    n_chunks = per_w // cg
    mesh = plsc.VectorSubcoreMesh(core_axis_name="c", subcore_axis_name="s")
    sds = jax.ShapeDtypeStruct((E, h), jnp.float32)

    @functools.partial(
        pl.kernel, mesh=mesh,
        out_type=[sds, sds],
        scratch_types=[
            pltpu.VMEM((cg,), jnp.int32),
            pltpu.VMEM((cg,), jnp.int32),
            pltpu.VMEM((cg, h), jnp.float32),
            pltpu.VMEM((cg, h), jnp.float32),
            pltpu.SemaphoreType.DMA,
            pltpu.SemaphoreType.DMA,
        ],
    )
    def k(u_hbm, v_hbm, src_hbm, dst_hbm, us_hbm, vd_hbm,
          sidx, didx, ubuf, vbuf, sem1, sem2):
        wid = lax.axis_index("s") * NC + lax.axis_index("c")
        base = wid * per_w

        def body(c, _):
            off = base + c * cg
            pltpu.sync_copy(src_hbm.at[pl.ds(off, cg)], sidx)
            pltpu.sync_copy(dst_hbm.at[pl.ds(off, cg)], didx)
            cp1 = pltpu.async_copy(u_hbm.at[sidx], ubuf, sem1)
            cp2 = pltpu.async_copy(v_hbm.at[didx], vbuf, sem2)
            cp1.wait()
            cp2.wait()
            pltpu.sync_copy(ubuf, us_hbm.at[pl.ds(off, cg)])
            pltpu.sync_copy(vbuf, vd_hbm.at[pl.ds(off, cg)])
            return ()

        lax.fori_loop(0, n_chunks, body, (), unroll=False)

    return k(u, v, src, dst)


def _layer(f, pos, src, dst, Wa, ba, Wb, bb):
    din = f.shape[1]
    waxT = Wa[:, :din].T
    wapT = Wa[:, din:].T
    wbT = Wb.T
    hh = Wa.shape[0]
    hp = max(hh, 128)
    if hp != hh:
        pad = ((0, 0), (0, hp - hh))
        waxT = jnp.pad(waxT, pad)
        wapT = jnp.pad(wapT, pad)
        ba = jnp.pad(ba, (0, hp - hh))
        wbT = jnp.pad(wbT, ((0, hp - hh), (0, 0)))
    u, v, s = _dense_node(f, pos, waxT, wapT, ba.reshape(1, -1), wbT,
                          bb.reshape(1, -1))
    us, vd = _edge_gather(u, v, src, dst)
    m = _edge_mlp(us, vd, wbT, bb.reshape(1, -1))
    agg = jax.ops.segment_max(m, dst, num_segments=f.shape[0])
    return jnp.maximum(agg, s)


def kernel(x, pos, edge_index, W1a, b1a, W1b, b1b, W2a, b2a, W2b, b2b,
           W3a, b3a, W3b, b3b):
    src = edge_index[0]
    dst = edge_index[1]
    h = _layer(x, pos, src, dst, W1a, b1a, W1b, b1b)
    h = _layer(h, pos, src, dst, W2a, b2a, W2b, b2b)
    h = _layer(h, pos, src, dst, W3a, b3a, W3b, b3b)
    return h
